# baseline (speedup 1.0000x reference)
import functools

import jax
import jax.numpy as jnp
from jax import lax
from jax.experimental import pallas as pl
from jax.experimental.pallas import tpu as pltpu
from jax.experimental.pallas import tpu_sc as plsc

_G = 1000
_GPAD = 1024
_BAND = 64


def _stage1(batch_ref, x_ref, w1_ref, b1_ref, w2_ref, b2_ref, e_ref, s_out_ref,
            s_ref):
    i = pl.program_id(0)
    nb = pl.num_programs(0)

    @pl.when(i == 0)
    def _init():
        s_ref[...] = jnp.zeros_like(s_ref)

    x = x_ref[...]
    bn = x.shape[0]
    w2 = w2_ref[...]
    h = jnp.tanh(
        jax.lax.dot_general(x, w1_ref[...], (((1,), (0,)), ((), ())),
                            preferred_element_type=jnp.float32)
        + b1_ref[...]
    )
    l = jax.lax.dot_general(h, w2, (((1,), (0,)), ((), ())),
                            preferred_element_type=jnp.float32) + b2_ref[0, 0]
    m = jnp.sum(jnp.abs(w2)) + jnp.abs(b2_ref[0, 0])
    e = jnp.exp(l - m)
    e_ref[...] = e
    eb = e.astype(jnp.bfloat16)

    g = batch_ref[0, 0, :]
    g0 = (batch_ref[0, 0, 0] // 8) * 8
    k = (batch_ref[0, 0, bn - 1] - g0) // _BAND + 1

    def body(b, _):
        start = g0 + b * _BAND
        rows = start + jax.lax.broadcasted_iota(jnp.int32, (_BAND, bn), 0)
        ohb = (rows == g[None, :]).astype(jnp.bfloat16)
        sc = jax.lax.dot_general(ohb, eb, (((1,), (0,)), ((), ())),
                                 preferred_element_type=jnp.float32)
        s_ref[pl.ds(start, _BAND), :] += sc
        return 0

    jax.lax.fori_loop(0, k, body, 0)

    @pl.when(i == nb - 1)
    def _fin():
        s_out_ref[...] = s_ref[:_G, :]


_C = 80
_NW = 32
_D = 128
_RPT = _GPAD // 16


def _sc_pool(x_hbm, e_hbm, batch_hbm, zeros_hbm, out_hbm, xv, ev, iv, shared):
    cid = lax.axis_index("c")
    sid = lax.axis_index("s")
    wid = sid * 2 + cid
    nchunks = x_hbm.shape[0] // _C

    pltpu.sync_copy(zeros_hbm, shared.at[pl.ds(sid * _RPT, _RPT)])
    plsc.subcore_barrier()

    def chunk_body(kk, _):
        c = wid + _NW * kk
        base = c * _C
        pltpu.sync_copy(x_hbm.at[pl.ds(base, _C)], xv)
        pltpu.sync_copy(e_hbm.at[pl.ds(base, _C)], ev)
        pltpu.sync_copy(batch_hbm.at[pl.ds(base, _C)], iv)
        for r0 in range(0, _C, 16):
            ev16 = ev[pl.ds(r0, 16)]
            for rr in range(16):
                er = ev16[rr]
                for j in range(_D // 16):
                    xv[r0 + rr, pl.ds(j * 16, 16)] = (
                        xv[r0 + rr, pl.ds(j * 16, 16)] * er)
        pltpu.sync_copy(xv, shared.at[iv], add=True)
        return 0

    nk = (nchunks - wid + _NW - 1) // _NW
    jax.lax.fori_loop(0, nk, chunk_body, 0)
    plsc.subcore_barrier()
    pltpu.sync_copy(shared.at[pl.ds(sid * _RPT, _RPT)],
                    out_hbm.at[cid, pl.ds(sid * _RPT, _RPT)])


def _stage3(p_ref, s_ref, out_ref):
    out_ref[...] = (p_ref[0, :_G, :] + p_ref[1, :_G, :]) / (s_ref[...] + 1e-16)


def kernel(x, batch, W1, b1, W2, b2):
    n, d = x.shape
    bn = 2000
    nb = n // bn
    batch_i32 = batch.astype(jnp.int32)
    batch3 = batch_i32.reshape(nb, 1, bn)
    b1r = b1.reshape(1, d)
    b2r = b2.reshape(1, 1)

    e, s = pl.pallas_call(
        _stage1,
        grid=(nb,),
        in_specs=[
            pl.BlockSpec((1, 1, bn), lambda i: (i, 0, 0)),
            pl.BlockSpec((bn, d), lambda i: (i, 0)),
            pl.BlockSpec((d, d), lambda i: (0, 0)),
            pl.BlockSpec((1, d), lambda i: (0, 0)),
            pl.BlockSpec((d, 1), lambda i: (0, 0)),
            pl.BlockSpec((1, 1), lambda i: (0, 0)),
        ],
        out_specs=[
            pl.BlockSpec((bn, 1), lambda i: (i, 0)),
            pl.BlockSpec((_G, 1), lambda i: (0, 0)),
        ],
        out_shape=[
            jax.ShapeDtypeStruct((n, 1), jnp.float32),
            jax.ShapeDtypeStruct((_G, 1), jnp.float32),
        ],
        scratch_shapes=[pltpu.VMEM((_GPAD, 1), jnp.float32)],
    )(batch3, x, W1, b1r, W2, b2r)

    mesh = plsc.VectorSubcoreMesh(core_axis_name="c", subcore_axis_name="s")
    sc_pool = pl.kernel(
        _sc_pool,
        out_type=jax.ShapeDtypeStruct((2, _GPAD, d), jnp.float32),
        mesh=mesh,
        scratch_types=[
            pltpu.VMEM((_C, d), jnp.float32),
            pltpu.VMEM((_C,), jnp.float32),
            pltpu.VMEM((_C,), jnp.int32),
            pltpu.VMEM_SHARED((_GPAD, d), jnp.float32),
        ],
    )
    zeros_hbm = jnp.zeros((_RPT, d), jnp.float32)
    partial = sc_pool(x, e.reshape(n), batch_i32, zeros_hbm)

    return pl.pallas_call(
        _stage3,
        out_shape=jax.ShapeDtypeStruct((_G, d), jnp.float32),
    )(partial, s)


# speedup vs baseline: 4.7943x; 4.7943x over previous
import jax
import jax.numpy as jnp
from jax.experimental import pallas as pl
from jax.experimental.pallas import tpu as pltpu

_G = 1000
_BAND = 64
_GPAD = ((_G - 1) // 8) * 8 + _BAND


def _fused(batch_ref, x_ref, w1_ref, b1_ref, w2_ref, b2_ref, out_ref,
           acc_ref, s_ref):
    i = pl.program_id(0)
    nb = pl.num_programs(0)

    @pl.when(i == 0)
    def _init():
        acc_ref[...] = jnp.zeros_like(acc_ref)
        s_ref[...] = jnp.zeros_like(s_ref)

    x = x_ref[...]
    bn = x.shape[0]
    w2 = w2_ref[...]
    h = jnp.tanh(
        jax.lax.dot_general(x, w1_ref[...], (((1,), (0,)), ((), ())),
                            preferred_element_type=jnp.float32)
        + b1_ref[...]
    )
    l = jax.lax.dot_general(h, w2, (((1,), (0,)), ((), ())),
                            preferred_element_type=jnp.float32) + b2_ref[0, 0]
    m = jnp.sum(jnp.abs(w2)) + jnp.abs(b2_ref[0, 0])
    e = jnp.exp(l - m)
    eb = e.astype(jnp.bfloat16)
    weighted = (x * e).astype(jnp.bfloat16)

    g = batch_ref[0, 0, :]
    g0 = (batch_ref[0, 0, 0] // 8) * 8
    k = (batch_ref[0, 0, bn - 1] - g0) // _BAND + 1

    def body(b, _):
        start = g0 + b * _BAND
        rows = start + jax.lax.broadcasted_iota(jnp.int32, (_BAND, bn), 0)
        ohb = (rows == g[None, :]).astype(jnp.bfloat16)
        contrib = jax.lax.dot_general(
            ohb, weighted, (((1,), (0,)), ((), ())),
            preferred_element_type=jnp.float32)
        acc_ref[pl.ds(start, _BAND), :] += contrib
        sc = jax.lax.dot_general(
            ohb, eb, (((1,), (0,)), ((), ())),
            preferred_element_type=jnp.float32)
        s_ref[pl.ds(start, _BAND), :] += sc
        return 0

    jax.lax.fori_loop(0, k, body, 0)

    @pl.when(i == nb - 1)
    def _fin():
        out_ref[...] = acc_ref[:_G, :] / (s_ref[:_G, :] + 1e-16)


def kernel(x, batch, W1, b1, W2, b2):
    n, d = x.shape
    bn = 4000
    nb = n // bn
    batch3 = batch.astype(jnp.int32).reshape(nb, 1, bn)
    b1r = b1.reshape(1, d)
    b2r = b2.reshape(1, 1)
    return pl.pallas_call(
        _fused,
        grid=(nb,),
        in_specs=[
            pl.BlockSpec((1, 1, bn), lambda i: (i, 0, 0)),
            pl.BlockSpec((bn, d), lambda i: (i, 0)),
            pl.BlockSpec((d, d), lambda i: (0, 0)),
            pl.BlockSpec((1, d), lambda i: (0, 0)),
            pl.BlockSpec((d, 1), lambda i: (0, 0)),
            pl.BlockSpec((1, 1), lambda i: (0, 0)),
        ],
        out_specs=pl.BlockSpec((_G, d), lambda i: (0, 0)),
        out_shape=jax.ShapeDtypeStruct((_G, d), jnp.float32),
        scratch_shapes=[
            pltpu.VMEM((_GPAD, d), jnp.float32),
            pltpu.VMEM((_GPAD, 1), jnp.float32),
        ],
    )(batch3, x, W1, b1r, W2, b2r)


# speedup vs baseline: 5.0171x; 1.0465x over previous
import jax
import jax.numpy as jnp
from jax.experimental import pallas as pl
from jax.experimental.pallas import tpu as pltpu

_G = 1000
_BAND = 128
_GPAD = ((_G - 1) // 8) * 8 + _BAND


def _fused(batch_ref, x_ref, w1_ref, b1_ref, w2_ref, b2_ref, out_ref,
           acc_ref, s_ref):
    i = pl.program_id(0)
    nb = pl.num_programs(0)

    @pl.when(i == 0)
    def _init():
        acc_ref[...] = jnp.zeros_like(acc_ref)
        s_ref[...] = jnp.zeros_like(s_ref)

    x = x_ref[...]
    bn = x.shape[0]
    w2 = w2_ref[...]
    h = jnp.tanh(
        jax.lax.dot_general(x, w1_ref[...], (((1,), (0,)), ((), ())),
                            preferred_element_type=jnp.float32)
        + b1_ref[...]
    )
    l = jax.lax.dot_general(h, w2, (((1,), (0,)), ((), ())),
                            preferred_element_type=jnp.float32) + b2_ref[0, 0]
    m = jnp.sum(jnp.abs(w2)) + jnp.abs(b2_ref[0, 0])
    e = jnp.exp(l - m)
    eb = e.astype(jnp.bfloat16)
    weighted = (x * e).astype(jnp.bfloat16)

    g = batch_ref[0, 0, :]
    g0 = (batch_ref[0, 0, 0] // 8) * 8
    k = (batch_ref[0, 0, bn - 1] - g0) // _BAND + 1

    def body(b, _):
        start = g0 + b * _BAND
        rows = start + jax.lax.broadcasted_iota(jnp.int32, (_BAND, bn), 0)
        ohb = (rows == g[None, :]).astype(jnp.bfloat16)
        contrib = jax.lax.dot_general(
            ohb, weighted, (((1,), (0,)), ((), ())),
            preferred_element_type=jnp.float32)
        acc_ref[pl.ds(start, _BAND), :] += contrib
        sc = jax.lax.dot_general(
            ohb, eb, (((1,), (0,)), ((), ())),
            preferred_element_type=jnp.float32)
        s_ref[pl.ds(start, _BAND), :] += sc
        return 0

    jax.lax.fori_loop(0, k, body, 0)

    @pl.when(i == nb - 1)
    def _fin():
        out_ref[...] = acc_ref[:_G, :] / (s_ref[:_G, :] + 1e-16)


def kernel(x, batch, W1, b1, W2, b2):
    n, d = x.shape
    bn = 10000
    nb = n // bn
    batch3 = batch.astype(jnp.int32).reshape(nb, 1, bn)
    b1r = b1.reshape(1, d)
    b2r = b2.reshape(1, 1)
    return pl.pallas_call(
        _fused,
        grid=(nb,),
        in_specs=[
            pl.BlockSpec((1, 1, bn), lambda i: (i, 0, 0)),
            pl.BlockSpec((bn, d), lambda i: (i, 0)),
            pl.BlockSpec((d, d), lambda i: (0, 0)),
            pl.BlockSpec((1, d), lambda i: (0, 0)),
            pl.BlockSpec((d, 1), lambda i: (0, 0)),
            pl.BlockSpec((1, 1), lambda i: (0, 0)),
        ],
        out_specs=pl.BlockSpec((_G, d), lambda i: (0, 0)),
        out_shape=jax.ShapeDtypeStruct((_G, d), jnp.float32),
        scratch_shapes=[
            pltpu.VMEM((_GPAD, d), jnp.float32),
            pltpu.VMEM((_GPAD, 1), jnp.float32),
        ],
    )(batch3, x, W1, b1r, W2, b2r)
